# Initial kernel scaffold; baseline (speedup 1.0000x reference)
import jax
import jax.numpy as jnp
from jax.experimental import pallas as pl


def kernel(x, edge_index, edge_attr, enc_W, enc_b, enc_g, enc_bt, ee_W, ee_b, eW1, eb1, eg1, ebt1, eW2, eb2, eg2, ebt2, nW1, nb1, ng1, nbt1, nW2, nb2, ng2, nbt2, dW1, db1, dW2, db2):
    raise NotImplementedError("write your pallas kernel here")


# speedup vs baseline: 3.4403x; 3.4403x over previous
import functools

import jax
import jax.numpy as jnp
from jax import lax
from jax.experimental import pallas as pl
from jax.experimental.pallas import tpu as pltpu
from jax.experimental.pallas import tpu_sc as plsc

N = 10000
E = 320000
H = 128

NC = 2
NS = 16
NW = NC * NS
EPW = E // NW
CH = 80
NIT = EPW // CH
NPAD = 10240
RPT = NPAD // NS

_mesh = plsc.VectorSubcoreMesh(
    core_axis_name="c", subcore_axis_name="s", num_cores=NC, num_subcores=NS)


def _ln(t, g, b):
    mu = jnp.mean(t, axis=-1, keepdims=True)
    t = t - mu
    var = jnp.mean(t * t, axis=-1, keepdims=True)
    return t * lax.rsqrt(var + 1e-5) * g + b


def _gelu(t):
    return t * 0.5 * (1.0 + lax.erf(t * 0.7071067811865476))


@functools.partial(
    pl.kernel,
    out_type=(jax.ShapeDtypeStruct((E, H), jnp.float32),
              jax.ShapeDtypeStruct((E, H), jnp.float32)),
    mesh=_mesh,
    scratch_types=[
        pltpu.VMEM((CH,), jnp.int32),
        pltpu.VMEM((CH,), jnp.int32),
        pltpu.VMEM((CH, H), jnp.float32),
        pltpu.VMEM((CH, H), jnp.float32),
        pltpu.SemaphoreType.DMA,
        pltpu.SemaphoreType.DMA,
    ],
)
def _sc_gather(h_hbm, row_hbm, col_hbm, hr_hbm, hc_hbm,
               idx_r, idx_c, buf_r, buf_c, sem_r, sem_c):
    wid = lax.axis_index("s") * NC + lax.axis_index("c")
    base0 = pl.multiple_of(wid * EPW, 8)

    def body(it, carry):
        base = pl.multiple_of(base0 + it * CH, 8)
        pltpu.sync_copy(row_hbm.at[pl.ds(base, CH)], idx_r)
        pltpu.sync_copy(col_hbm.at[pl.ds(base, CH)], idx_c)
        cr = pltpu.async_copy(h_hbm.at[idx_r], buf_r, sem_r)
        cc = pltpu.async_copy(h_hbm.at[idx_c], buf_c, sem_c)
        cr.wait()
        cc.wait()
        pltpu.sync_copy(buf_r, hr_hbm.at[pl.ds(base, CH)])
        pltpu.sync_copy(buf_c, hc_hbm.at[pl.ds(base, CH)])
        return carry

    lax.fori_loop(0, NIT, body, 0)


@functools.partial(
    pl.kernel,
    out_type=jax.ShapeDtypeStruct((NC, NPAD, H), jnp.float32),
    mesh=_mesh,
    scratch_types=[
        pltpu.VMEM((CH,), jnp.int32),
        pltpu.VMEM((CH, H), jnp.float32),
        pltpu.VMEM_SHARED((NPAD, H), jnp.float32),
    ],
)
def _sc_scatter_add(ea_hbm, col_hbm, zeros_hbm, out_hbm, idx_v, buf_v, acc_sh):
    cid = lax.axis_index("c")
    sid = lax.axis_index("s")
    wid = sid * NC + cid
    base0 = pl.multiple_of(wid * EPW, 8)
    rbase = pl.multiple_of(sid * RPT, 1)

    pltpu.sync_copy(zeros_hbm.at[pl.ds(rbase, RPT)], acc_sh.at[pl.ds(rbase, RPT)])
    plsc.subcore_barrier()

    def body(it, carry):
        base = pl.multiple_of(base0 + it * CH, 8)
        pltpu.sync_copy(col_hbm.at[pl.ds(base, CH)], idx_v)
        pltpu.sync_copy(ea_hbm.at[pl.ds(base, CH)], buf_v)
        pltpu.sync_copy(buf_v, acc_sh.at[idx_v], add=True)
        return carry

    lax.fori_loop(0, NIT, body, 0)
    plsc.subcore_barrier()
    pltpu.sync_copy(acc_sh.at[pl.ds(rbase, RPT)],
                    out_hbm.at[cid, pl.ds(rbase, RPT)])


@functools.partial(
    pl.kernel,
    out_type=jax.ShapeDtypeStruct((NC, NPAD, H), jnp.float32),
    mesh=_mesh,
    scratch_types=[
        pltpu.VMEM((CH,), jnp.int32),
        pltpu.VMEM((CH, H), jnp.float32),
        pltpu.VMEM_SHARED((NPAD, H), jnp.float32),
    ],
)
def _sc_degree(col_hbm, ones_hbm, zeros_hbm, out_hbm, idx_v, buf_v, acc_sh):
    cid = lax.axis_index("c")
    sid = lax.axis_index("s")
    wid = sid * NC + cid
    base0 = pl.multiple_of(wid * EPW, 8)
    rbase = pl.multiple_of(sid * RPT, 1)

    pltpu.sync_copy(zeros_hbm.at[pl.ds(rbase, RPT)], acc_sh.at[pl.ds(rbase, RPT)])
    pltpu.sync_copy(ones_hbm, buf_v)
    plsc.subcore_barrier()

    def body(it, carry):
        base = pl.multiple_of(base0 + it * CH, 8)
        pltpu.sync_copy(col_hbm.at[pl.ds(base, CH)], idx_v)
        pltpu.sync_copy(buf_v, acc_sh.at[idx_v], add=True)
        return carry

    lax.fori_loop(0, NIT, body, 0)
    plsc.subcore_barrier()
    pltpu.sync_copy(acc_sh.at[pl.ds(rbase, RPT)],
                    out_hbm.at[cid, pl.ds(rbase, RPT)])


BE = 2560
BN = 2000


def _enc_body(x_ref, w_ref, b_ref, g_ref, bt_ref, o_ref):
    t = jnp.dot(x_ref[...], w_ref[...], preferred_element_type=jnp.float32)
    t = t + b_ref[...]
    o_ref[...] = _gelu(_ln(t, g_ref[...], bt_ref[...]))


def _eenc_body(a_ref, w_ref, b_ref, o_ref):
    o_ref[...] = jnp.dot(a_ref[...], w_ref[...],
                         preferred_element_type=jnp.float32) + b_ref[...]


def _edge_body(hr_ref, hc_ref, ea_ref, wr_ref, wc_ref, we_ref, b1_ref,
               g1_ref, bt1_ref, w2_ref, b2_ref, g2_ref, bt2_ref, o_ref):
    ea = ea_ref[...]
    t = (jnp.dot(hr_ref[...], wr_ref[...], preferred_element_type=jnp.float32)
         + jnp.dot(hc_ref[...], wc_ref[...], preferred_element_type=jnp.float32)
         + jnp.dot(ea, we_ref[...], preferred_element_type=jnp.float32)
         + b1_ref[...])
    t = _gelu(_ln(t, g1_ref[...], bt1_ref[...]))
    m = jnp.dot(t, w2_ref[...], preferred_element_type=jnp.float32) + b2_ref[...]
    m = _ln(m, g2_ref[...], bt2_ref[...])
    o_ref[...] = ea + m


def _node_body(h_ref, p_ref, inv_ref, w1a_ref, w1b_ref, b1_ref, g1_ref,
               bt1_ref, w2_ref, b2_ref, g2_ref, bt2_ref, o_ref):
    h = h_ref[...]
    agg = (p_ref[0] + p_ref[1]) * inv_ref[...]
    t = (jnp.dot(h, w1a_ref[...], preferred_element_type=jnp.float32)
         + jnp.dot(agg, w1b_ref[...], preferred_element_type=jnp.float32)
         + b1_ref[...])
    t = _gelu(_ln(t, g1_ref[...], bt1_ref[...]))
    u = jnp.dot(t, w2_ref[...], preferred_element_type=jnp.float32) + b2_ref[...]
    u = _ln(u, g2_ref[...], bt2_ref[...])
    o_ref[...] = h + u


def _inv_body(c_ref, o_ref):
    o_ref[...] = 1.0 / jnp.maximum(c_ref[0] + c_ref[1], 1.0)


def _dec_body(h_ref, w1_ref, b1_ref, w2_ref, b2_ref, o_ref):
    t = _gelu(jnp.dot(h_ref[...], w1_ref[...],
                      preferred_element_type=jnp.float32) + b1_ref[...])
    o_ref[...] = jnp.dot(t, w2_ref[...],
                         preferred_element_type=jnp.float32) + b2_ref[...]


def _rowspec(b, w):
    return pl.BlockSpec((b, w), lambda i: (i, 0))


def _wspec(r, c):
    return pl.BlockSpec((r, c), lambda i: (0, 0))


def _tc_call(body, grid, in_specs, out_spec, out_shape):
    return pl.pallas_call(
        body,
        grid=(grid,),
        in_specs=in_specs,
        out_specs=out_spec,
        out_shape=out_shape,
    )


def kernel(x, edge_index, edge_attr, enc_W, enc_b, enc_g, enc_bt, ee_W, ee_b,
           eW1, eb1, eg1, ebt1, eW2, eb2, eg2, ebt2, nW1, nb1, ng1, nbt1,
           nW2, nb2, ng2, nbt2, dW1, db1, dW2, db2):
    f32 = jnp.float32
    row = edge_index[0]
    col = edge_index[1]
    zeros_n = jnp.zeros((NPAD, H), f32)
    ones_ch = jnp.ones((CH, H), f32)

    h = _tc_call(
        _enc_body, N // BN,
        [_rowspec(BN, H), _wspec(H, H), _wspec(1, H), _wspec(1, H), _wspec(1, H)],
        _rowspec(BN, H), jax.ShapeDtypeStruct((N, H), f32),
    )(x, enc_W, enc_b.reshape(1, H), enc_g.reshape(1, H), enc_bt.reshape(1, H))

    ea = _tc_call(
        _eenc_body, E // BE,
        [_rowspec(BE, 16), _wspec(16, H), _wspec(1, H)],
        _rowspec(BE, H), jax.ShapeDtypeStruct((E, H), f32),
    )(edge_attr, ee_W, ee_b.reshape(1, H))

    deg = _sc_degree(col, ones_ch, zeros_n)
    inv_cnt = _tc_call(
        _inv_body, N // BN,
        [pl.BlockSpec((NC, BN, H), lambda i: (0, i, 0))],
        _rowspec(BN, H), jax.ShapeDtypeStruct((N, H), f32),
    )(deg)

    for i in range(eW1.shape[0]):
        w1 = eW1[i]
        wr, wc, we = w1[:H], w1[H:2 * H], w1[2 * H:]

        hr, hc = _sc_gather(h, row, col)

        ea = _tc_call(
            _edge_body, E // BE,
            [_rowspec(BE, H), _rowspec(BE, H), _rowspec(BE, H),
             _wspec(H, 2 * H), _wspec(H, 2 * H), _wspec(H, 2 * H),
             _wspec(1, 2 * H), _wspec(1, 2 * H), _wspec(1, 2 * H),
             _wspec(2 * H, H), _wspec(1, H), _wspec(1, H), _wspec(1, H)],
            _rowspec(BE, H), jax.ShapeDtypeStruct((E, H), f32),
        )(hr, hc, ea, wr, wc, we,
          eb1[i].reshape(1, 2 * H), eg1[i].reshape(1, 2 * H),
          ebt1[i].reshape(1, 2 * H), eW2[i], eb2[i].reshape(1, H),
          eg2[i].reshape(1, H), ebt2[i].reshape(1, H))

        parts = _sc_scatter_add(ea, col, zeros_n)

        nw1 = nW1[i]
        h = _tc_call(
            _node_body, N // BN,
            [_rowspec(BN, H), pl.BlockSpec((NC, BN, H), lambda i: (0, i, 0)),
             _rowspec(BN, H),
             _wspec(H, 2 * H), _wspec(H, 2 * H),
             _wspec(1, 2 * H), _wspec(1, 2 * H), _wspec(1, 2 * H),
             _wspec(2 * H, H), _wspec(1, H), _wspec(1, H), _wspec(1, H)],
            _rowspec(BN, H), jax.ShapeDtypeStruct((N, H), f32),
        )(h, parts, inv_cnt, nw1[:H], nw1[H:],
          nb1[i].reshape(1, 2 * H), ng1[i].reshape(1, 2 * H),
          nbt1[i].reshape(1, 2 * H), nW2[i], nb2[i].reshape(1, H),
          ng2[i].reshape(1, H), nbt2[i].reshape(1, H))

    dW2p = jnp.zeros((H, H), f32).at[:, :dW2.shape[1]].set(dW2)
    db2p = jnp.zeros((1, H), f32).at[0, :db2.shape[0]].set(db2)
    out = _tc_call(
        _dec_body, N // BN,
        [_rowspec(BN, H), _wspec(H, H), _wspec(1, H), _wspec(H, H), _wspec(1, H)],
        _rowspec(BN, H), jax.ShapeDtypeStruct((N, H), f32),
    )(h, dW1, db1.reshape(1, H), dW2p, db2p)
    return out[:, :dW2.shape[1]]


# speedup vs baseline: 5.2923x; 1.5383x over previous
import functools

import jax
import jax.numpy as jnp
from jax import lax
from jax.experimental import pallas as pl
from jax.experimental.pallas import tpu as pltpu
from jax.experimental.pallas import tpu_sc as plsc

N = 10000
E = 320000
H = 128

NC = 2
NS = 16
NW = NC * NS
CH = 80
NPAD = 10240
RPT = NPAD // NS

EA = 163840
EB = E - EA

_mesh = plsc.VectorSubcoreMesh(
    core_axis_name="c", subcore_axis_name="s", num_cores=NC, num_subcores=NS)


def _ln(t, g, b):
    mu = jnp.mean(t, axis=-1, keepdims=True)
    t = t - mu
    var = jnp.mean(t * t, axis=-1, keepdims=True)
    return t * lax.rsqrt(var + 1e-5) * g + b


def _gelu(t):
    return t * 0.5 * (1.0 + lax.erf(t * 0.7071067811865476))


def _make_gather(ne):
    epw = ne // NW
    nit = epw // CH

    @functools.partial(
        pl.kernel,
        out_type=(jax.ShapeDtypeStruct((ne, H), jnp.float32),
                  jax.ShapeDtypeStruct((ne, H), jnp.float32)),
        mesh=_mesh,
        scratch_types=[
            pltpu.VMEM((2, CH), jnp.int32),
            pltpu.VMEM((2, CH), jnp.int32),
            pltpu.VMEM((2, CH, H), jnp.float32),
            pltpu.VMEM((2, CH, H), jnp.float32),
            pltpu.SemaphoreType.DMA((2,)),
            pltpu.SemaphoreType.DMA((2,)),
            pltpu.SemaphoreType.DMA((2,)),
            pltpu.SemaphoreType.DMA((2,)),
            pltpu.SemaphoreType.DMA((2,)),
            pltpu.SemaphoreType.DMA((2,)),
        ],
    )
    def gather(h_hbm, row_hbm, col_hbm, hr_hbm, hc_hbm,
               idx_r, idx_c, buf_r, buf_c, sem_r, sem_c, wsem_r, wsem_c,
               isem_r, isem_c):
        wid = lax.axis_index("s") * NC + lax.axis_index("c")
        base0 = pl.multiple_of(wid * epw, 8)

        def start_idx(k, s):
            base = pl.multiple_of(base0 + k * CH, 8)
            pltpu.async_copy(row_hbm.at[pl.ds(base, CH)], idx_r.at[s],
                             isem_r.at[s])
            pltpu.async_copy(col_hbm.at[pl.ds(base, CH)], idx_c.at[s],
                             isem_c.at[s])

        def wait_idx(s):
            pltpu.make_async_copy(row_hbm.at[pl.ds(0, CH)], idx_r.at[s],
                                  isem_r.at[s]).wait()
            pltpu.make_async_copy(col_hbm.at[pl.ds(0, CH)], idx_c.at[s],
                                  isem_c.at[s]).wait()

        def start_gather(s):
            pltpu.async_copy(h_hbm.at[idx_r.at[s]], buf_r.at[s], sem_r.at[s])
            pltpu.async_copy(h_hbm.at[idx_c.at[s]], buf_c.at[s], sem_c.at[s])

        def wait_wb(s):
            pltpu.make_async_copy(buf_r.at[s], hr_hbm.at[pl.ds(0, CH)],
                                  wsem_r.at[s]).wait()
            pltpu.make_async_copy(buf_c.at[s], hc_hbm.at[pl.ds(0, CH)],
                                  wsem_c.at[s]).wait()

        start_idx(0, 0)
        if nit > 1:
            start_idx(1, 1)
        wait_idx(0)
        start_gather(0)

        def body(it, carry):
            s = lax.rem(it, 2)
            n = 1 - s

            @pl.when(it + 1 < nit)
            def _():
                @pl.when(it >= 1)
                def _():
                    wait_wb(n)
                wait_idx(n)
                start_gather(n)

            base = pl.multiple_of(base0 + it * CH, 8)
            pltpu.make_async_copy(hr_hbm.at[pl.ds(0, CH)], buf_r.at[s],
                                  sem_r.at[s]).wait()
            pltpu.make_async_copy(hr_hbm.at[pl.ds(0, CH)], buf_c.at[s],
                                  sem_c.at[s]).wait()

            @pl.when(it + 2 < nit)
            def _():
                start_idx(it + 2, s)

            pltpu.async_copy(buf_r.at[s], hr_hbm.at[pl.ds(base, CH)],
                             wsem_r.at[s])
            pltpu.async_copy(buf_c.at[s], hc_hbm.at[pl.ds(base, CH)],
                             wsem_c.at[s])
            return carry

        lax.fori_loop(0, nit, body, 0)
        wait_wb(lax.rem(nit - 1, 2))
        if nit >= 2:
            wait_wb(lax.rem(nit, 2))

    return gather


def _make_scatter(ne):
    epw = ne // NW
    nit = epw // CH

    @functools.partial(
        pl.kernel,
        out_type=jax.ShapeDtypeStruct((NC, NPAD, H), jnp.float32),
        mesh=_mesh,
        scratch_types=[
            pltpu.VMEM((2, CH), jnp.int32),
            pltpu.VMEM((2, CH, H), jnp.float32),
            pltpu.VMEM_SHARED((NPAD, H), jnp.float32),
            pltpu.SemaphoreType.DMA((2,)),
            pltpu.SemaphoreType.DMA((2,)),
            pltpu.SemaphoreType.DMA((2,)),
        ],
    )
    def scatter(ea_hbm, col_hbm, zeros_hbm, out_hbm, idx_v, buf_v, acc_sh,
                sem, asem, isem):
        cid = lax.axis_index("c")
        sid = lax.axis_index("s")
        wid = sid * NC + cid
        base0 = pl.multiple_of(wid * epw, 8)
        rbase = pl.multiple_of(sid * RPT, 1)

        pltpu.sync_copy(zeros_hbm.at[pl.ds(rbase, RPT)],
                        acc_sh.at[pl.ds(rbase, RPT)])
        plsc.subcore_barrier()

        def start(k, s):
            base = pl.multiple_of(base0 + k * CH, 8)
            pltpu.async_copy(col_hbm.at[pl.ds(base, CH)], idx_v.at[s],
                             isem.at[s])
            pltpu.async_copy(ea_hbm.at[pl.ds(base, CH)], buf_v.at[s],
                             sem.at[s])

        start(0, 0)

        def wait_add(s):
            pltpu.make_async_copy(buf_v.at[s], acc_sh.at[pl.ds(0, CH)],
                                  asem.at[s]).wait()

        def body(it, carry):
            s = lax.rem(it, 2)
            n = 1 - s

            @pl.when(it + 1 < nit)
            def _():
                @pl.when(it >= 1)
                def _():
                    wait_add(n)
                start(it + 1, n)

            pltpu.make_async_copy(ea_hbm.at[pl.ds(0, CH)], buf_v.at[s],
                                  sem.at[s]).wait()
            pltpu.make_async_copy(col_hbm.at[pl.ds(0, CH)], idx_v.at[s],
                                  isem.at[s]).wait()
            pltpu.async_copy(buf_v.at[s], acc_sh.at[idx_v.at[s]], asem.at[s],
                             add=True)
            return carry

        lax.fori_loop(0, nit, body, 0)
        wait_add(lax.rem(nit - 1, 2))
        if nit >= 2:
            wait_add(lax.rem(nit, 2))
        plsc.subcore_barrier()
        pltpu.sync_copy(acc_sh.at[pl.ds(rbase, RPT)],
                        out_hbm.at[cid, pl.ds(rbase, RPT)])

    return scatter


def _make_degree():
    epw = E // NW
    nit = epw // CH

    @functools.partial(
        pl.kernel,
        out_type=jax.ShapeDtypeStruct((NC, NPAD, H), jnp.float32),
        mesh=_mesh,
        scratch_types=[
            pltpu.VMEM((2, CH), jnp.int32),
            pltpu.VMEM((CH, H), jnp.float32),
            pltpu.VMEM_SHARED((NPAD, H), jnp.float32),
            pltpu.SemaphoreType.DMA((2,)),
            pltpu.SemaphoreType.DMA((2,)),
        ],
    )
    def degree(col_hbm, ones_hbm, zeros_hbm, out_hbm, idx_v, buf_v, acc_sh,
               asem, isem):
        cid = lax.axis_index("c")
        sid = lax.axis_index("s")
        wid = sid * NC + cid
        base0 = pl.multiple_of(wid * epw, 8)
        rbase = pl.multiple_of(sid * RPT, 1)

        pltpu.sync_copy(zeros_hbm.at[pl.ds(rbase, RPT)],
                        acc_sh.at[pl.ds(rbase, RPT)])
        pltpu.sync_copy(ones_hbm, buf_v)
        plsc.subcore_barrier()

        def start_idx(k, s):
            base = pl.multiple_of(base0 + k * CH, 8)
            pltpu.async_copy(col_hbm.at[pl.ds(base, CH)], idx_v.at[s],
                             isem.at[s])

        def wait_add(s):
            pltpu.make_async_copy(buf_v, acc_sh.at[pl.ds(0, CH)],
                                  asem.at[s]).wait()

        start_idx(0, 0)

        def body(it, carry):
            s = lax.rem(it, 2)
            n = 1 - s

            @pl.when(it + 1 < nit)
            def _():
                @pl.when(it >= 1)
                def _():
                    wait_add(n)
                start_idx(it + 1, n)

            pltpu.make_async_copy(col_hbm.at[pl.ds(0, CH)], idx_v.at[s],
                                  isem.at[s]).wait()
            pltpu.async_copy(buf_v, acc_sh.at[idx_v.at[s]], asem.at[s],
                             add=True)
            return carry

        lax.fori_loop(0, nit, body, 0)
        wait_add(lax.rem(nit - 1, 2))
        if nit >= 2:
            wait_add(lax.rem(nit, 2))
        plsc.subcore_barrier()
        pltpu.sync_copy(acc_sh.at[pl.ds(rbase, RPT)],
                        out_hbm.at[cid, pl.ds(rbase, RPT)])

    return degree


_gather_a = _make_gather(EA)
_gather_b = _make_gather(EB)
_scatter_a = _make_scatter(EA)
_scatter_b = _make_scatter(EB)
_degree = _make_degree()


BE = 2560
BN = 2000


def _enc_body(x_ref, w_ref, b_ref, g_ref, bt_ref, o_ref):
    t = jnp.dot(x_ref[...], w_ref[...], preferred_element_type=jnp.float32)
    t = t + b_ref[...]
    o_ref[...] = _gelu(_ln(t, g_ref[...], bt_ref[...]))


def _eenc_body(a_ref, w_ref, b_ref, o_ref):
    o_ref[...] = jnp.dot(a_ref[...], w_ref[...],
                         preferred_element_type=jnp.float32) + b_ref[...]


def _edge_body(hr_ref, hc_ref, ea_ref, wr_ref, wc_ref, we_ref, b1_ref,
               g1_ref, bt1_ref, w2_ref, b2_ref, g2_ref, bt2_ref, o_ref):
    ea = ea_ref[...]
    t = (jnp.dot(hr_ref[...].astype(jnp.bfloat16), wr_ref[...],
                 preferred_element_type=jnp.float32)
         + jnp.dot(hc_ref[...].astype(jnp.bfloat16), wc_ref[...],
                   preferred_element_type=jnp.float32)
         + jnp.dot(ea.astype(jnp.bfloat16), we_ref[...],
                   preferred_element_type=jnp.float32)
         + b1_ref[...])
    t = _gelu(_ln(t, g1_ref[...], bt1_ref[...]))
    m = jnp.dot(t.astype(jnp.bfloat16), w2_ref[...],
                preferred_element_type=jnp.float32) + b2_ref[...]
    m = _ln(m, g2_ref[...], bt2_ref[...])
    o_ref[...] = ea + m


def _node_body(h_ref, pa_ref, pb_ref, inv_ref, w1a_ref, w1b_ref, b1_ref,
               g1_ref, bt1_ref, w2_ref, b2_ref, g2_ref, bt2_ref, o_ref):
    h = h_ref[...]
    agg = (pa_ref[0] + pa_ref[1] + pb_ref[0] + pb_ref[1]) * inv_ref[...]
    t = (jnp.dot(h, w1a_ref[...], preferred_element_type=jnp.float32)
         + jnp.dot(agg, w1b_ref[...], preferred_element_type=jnp.float32)
         + b1_ref[...])
    t = _gelu(_ln(t, g1_ref[...], bt1_ref[...]))
    u = jnp.dot(t, w2_ref[...], preferred_element_type=jnp.float32) + b2_ref[...]
    u = _ln(u, g2_ref[...], bt2_ref[...])
    o_ref[...] = h + u


def _inv_body(c_ref, o_ref):
    o_ref[...] = 1.0 / jnp.maximum(c_ref[0] + c_ref[1], 1.0)


def _dec_body(h_ref, w1_ref, b1_ref, w2_ref, b2_ref, o_ref):
    t = _gelu(jnp.dot(h_ref[...], w1_ref[...],
                      preferred_element_type=jnp.float32) + b1_ref[...])
    o_ref[...] = jnp.dot(t, w2_ref[...],
                         preferred_element_type=jnp.float32) + b2_ref[...]


def _rowspec(b, w):
    return pl.BlockSpec((b, w), lambda i: (i, 0))


def _wspec(r, c):
    return pl.BlockSpec((r, c), lambda i: (0, 0))


def _tc_call(body, grid, in_specs, out_spec, out_shape):
    return pl.pallas_call(
        body,
        grid=(grid,),
        in_specs=in_specs,
        out_specs=out_spec,
        out_shape=out_shape,
    )


def _edge_mlp(hr, hc, ea, wr, wc, we, b1, g1, bt1, w2, b2, g2, bt2):
    ne = ea.shape[0]
    return _tc_call(
        _edge_body, ne // BE,
        [_rowspec(BE, H), _rowspec(BE, H), _rowspec(BE, H),
         _wspec(H, 2 * H), _wspec(H, 2 * H), _wspec(H, 2 * H),
         _wspec(1, 2 * H), _wspec(1, 2 * H), _wspec(1, 2 * H),
         _wspec(2 * H, H), _wspec(1, H), _wspec(1, H), _wspec(1, H)],
        _rowspec(BE, H), jax.ShapeDtypeStruct((ne, H), jnp.float32),
    )(hr, hc, ea, wr, wc, we, b1, g1, bt1, w2, b2, g2, bt2)


def kernel(x, edge_index, edge_attr, enc_W, enc_b, enc_g, enc_bt, ee_W, ee_b,
           eW1, eb1, eg1, ebt1, eW2, eb2, eg2, ebt2, nW1, nb1, ng1, nbt1,
           nW2, nb2, ng2, nbt2, dW1, db1, dW2, db2):
    f32 = jnp.float32
    bf16 = jnp.bfloat16
    row = edge_index[0]
    col = edge_index[1]
    row_a, row_b = row[:EA], row[EA:]
    col_a, col_b = col[:EA], col[EA:]
    zeros_n = jnp.zeros((NPAD, H), f32)
    ones_ch = jnp.ones((CH, H), f32)

    h = _tc_call(
        _enc_body, N // BN,
        [_rowspec(BN, H), _wspec(H, H), _wspec(1, H), _wspec(1, H), _wspec(1, H)],
        _rowspec(BN, H), jax.ShapeDtypeStruct((N, H), f32),
    )(x, enc_W, enc_b.reshape(1, H), enc_g.reshape(1, H), enc_bt.reshape(1, H))

    ea_a = _tc_call(
        _eenc_body, EA // BE,
        [_rowspec(BE, 16), _wspec(16, H), _wspec(1, H)],
        _rowspec(BE, H), jax.ShapeDtypeStruct((EA, H), f32),
    )(edge_attr[:EA], ee_W, ee_b.reshape(1, H))
    ea_b = _tc_call(
        _eenc_body, EB // BE,
        [_rowspec(BE, 16), _wspec(16, H), _wspec(1, H)],
        _rowspec(BE, H), jax.ShapeDtypeStruct((EB, H), f32),
    )(edge_attr[EA:], ee_W, ee_b.reshape(1, H))

    deg = _degree(col, ones_ch, zeros_n)
    inv_cnt = _tc_call(
        _inv_body, N // BN,
        [pl.BlockSpec((NC, BN, H), lambda i: (0, i, 0))],
        _rowspec(BN, H), jax.ShapeDtypeStruct((N, H), f32),
    )(deg)

    for i in range(eW1.shape[0]):
        w1 = eW1[i]
        wr = w1[:H].astype(bf16)
        wc = w1[H:2 * H].astype(bf16)
        we = w1[2 * H:].astype(bf16)
        eb1i = eb1[i].reshape(1, 2 * H)
        eg1i = eg1[i].reshape(1, 2 * H)
        ebt1i = ebt1[i].reshape(1, 2 * H)
        ew2i = eW2[i].astype(bf16)
        eb2i = eb2[i].reshape(1, H)
        eg2i = eg2[i].reshape(1, H)
        ebt2i = ebt2[i].reshape(1, H)

        hr_a, hc_a = _gather_a(h, row_a, col_a)
        hr_b, hc_b = _gather_b(h, row_b, col_b)
        ea_a = _edge_mlp(hr_a, hc_a, ea_a, wr, wc, we, eb1i, eg1i, ebt1i,
                         ew2i, eb2i, eg2i, ebt2i)
        pa = _scatter_a(ea_a, col_a, zeros_n)
        ea_b = _edge_mlp(hr_b, hc_b, ea_b, wr, wc, we, eb1i, eg1i, ebt1i,
                         ew2i, eb2i, eg2i, ebt2i)
        pb = _scatter_b(ea_b, col_b, zeros_n)

        nw1 = nW1[i]
        h = _tc_call(
            _node_body, N // BN,
            [_rowspec(BN, H),
             pl.BlockSpec((NC, BN, H), lambda i: (0, i, 0)),
             pl.BlockSpec((NC, BN, H), lambda i: (0, i, 0)),
             _rowspec(BN, H),
             _wspec(H, 2 * H), _wspec(H, 2 * H),
             _wspec(1, 2 * H), _wspec(1, 2 * H), _wspec(1, 2 * H),
             _wspec(2 * H, H), _wspec(1, H), _wspec(1, H), _wspec(1, H)],
            _rowspec(BN, H), jax.ShapeDtypeStruct((N, H), f32),
        )(h, pa, pb, inv_cnt, nw1[:H], nw1[H:],
          nb1[i].reshape(1, 2 * H), ng1[i].reshape(1, 2 * H),
          nbt1[i].reshape(1, 2 * H), nW2[i], nb2[i].reshape(1, H),
          ng2[i].reshape(1, H), nbt2[i].reshape(1, H))

    dW2p = jnp.zeros((H, H), f32).at[:, :dW2.shape[1]].set(dW2)
    db2p = jnp.zeros((1, H), f32).at[0, :db2.shape[0]].set(db2)
    out = _tc_call(
        _dec_body, N // BN,
        [_rowspec(BN, H), _wspec(H, H), _wspec(1, H), _wspec(H, H), _wspec(1, H)],
        _rowspec(BN, H), jax.ShapeDtypeStruct((N, H), f32),
    )(h, dW1, db1.reshape(1, H), dW2p, db2p)
    return out[:, :dW2.shape[1]]


# speedup vs baseline: 5.4848x; 1.0364x over previous
import functools

import jax
import jax.numpy as jnp
from jax import lax
from jax.experimental import pallas as pl
from jax.experimental.pallas import tpu as pltpu
from jax.experimental.pallas import tpu_sc as plsc

N = 10000
E = 320000
H = 128

NC = 2
NS = 16
NW = NC * NS
CH = 80
NPAD = 10240
RPT = NPAD // NS

EA = 163840
EB = E - EA

_mesh = plsc.VectorSubcoreMesh(
    core_axis_name="c", subcore_axis_name="s", num_cores=NC, num_subcores=NS)


def _ln(t, g, b):
    mu = jnp.mean(t, axis=-1, keepdims=True)
    t = t - mu
    var = jnp.mean(t * t, axis=-1, keepdims=True)
    return t * lax.rsqrt(var + 1e-5) * g + b


def _gelu(t):
    return t * 0.5 * (1.0 + lax.erf(t * 0.7071067811865476))


def _make_gather(ne):
    epw = ne // NW
    nit = epw // CH

    @functools.partial(
        pl.kernel,
        out_type=(jax.ShapeDtypeStruct((ne, H), jnp.float32),
                  jax.ShapeDtypeStruct((ne, H), jnp.float32)),
        mesh=_mesh,
        scratch_types=[
            pltpu.VMEM((2, CH), jnp.int32),
            pltpu.VMEM((2, CH), jnp.int32),
            pltpu.VMEM((2, CH, H), jnp.float32),
            pltpu.VMEM((2, CH, H), jnp.float32),
            pltpu.SemaphoreType.DMA((2,)),
            pltpu.SemaphoreType.DMA((2,)),
            pltpu.SemaphoreType.DMA((2,)),
            pltpu.SemaphoreType.DMA((2,)),
            pltpu.SemaphoreType.DMA((2,)),
            pltpu.SemaphoreType.DMA((2,)),
        ],
    )
    def gather(h_hbm, row_hbm, col_hbm, hr_hbm, hc_hbm,
               idx_r, idx_c, buf_r, buf_c, sem_r, sem_c, wsem_r, wsem_c,
               isem_r, isem_c):
        wid = lax.axis_index("s") * NC + lax.axis_index("c")
        base0 = pl.multiple_of(wid * epw, 8)

        def start_idx(k, s):
            base = pl.multiple_of(base0 + k * CH, 8)
            pltpu.async_copy(row_hbm.at[pl.ds(base, CH)], idx_r.at[s],
                             isem_r.at[s])
            pltpu.async_copy(col_hbm.at[pl.ds(base, CH)], idx_c.at[s],
                             isem_c.at[s])

        def wait_idx(s):
            pltpu.make_async_copy(row_hbm.at[pl.ds(0, CH)], idx_r.at[s],
                                  isem_r.at[s]).wait()
            pltpu.make_async_copy(col_hbm.at[pl.ds(0, CH)], idx_c.at[s],
                                  isem_c.at[s]).wait()

        def start_gather(s):
            pltpu.async_copy(h_hbm.at[idx_r.at[s]], buf_r.at[s], sem_r.at[s])
            pltpu.async_copy(h_hbm.at[idx_c.at[s]], buf_c.at[s], sem_c.at[s])

        def wait_wb(s):
            pltpu.make_async_copy(buf_r.at[s], hr_hbm.at[pl.ds(0, CH)],
                                  wsem_r.at[s]).wait()
            pltpu.make_async_copy(buf_c.at[s], hc_hbm.at[pl.ds(0, CH)],
                                  wsem_c.at[s]).wait()

        start_idx(0, 0)
        if nit > 1:
            start_idx(1, 1)
        wait_idx(0)
        start_gather(0)

        def body(it, carry):
            s = lax.rem(it, 2)
            n = 1 - s

            @pl.when(it + 1 < nit)
            def _():
                @pl.when(it >= 1)
                def _():
                    wait_wb(n)
                wait_idx(n)
                start_gather(n)

            base = pl.multiple_of(base0 + it * CH, 8)
            pltpu.make_async_copy(hr_hbm.at[pl.ds(0, CH)], buf_r.at[s],
                                  sem_r.at[s]).wait()
            pltpu.make_async_copy(hr_hbm.at[pl.ds(0, CH)], buf_c.at[s],
                                  sem_c.at[s]).wait()

            @pl.when(it + 2 < nit)
            def _():
                start_idx(it + 2, s)

            pltpu.async_copy(buf_r.at[s], hr_hbm.at[pl.ds(base, CH)],
                             wsem_r.at[s])
            pltpu.async_copy(buf_c.at[s], hc_hbm.at[pl.ds(base, CH)],
                             wsem_c.at[s])
            return carry

        lax.fori_loop(0, nit, body, 0)
        wait_wb(lax.rem(nit - 1, 2))
        if nit >= 2:
            wait_wb(lax.rem(nit, 2))

    return gather


def _make_scatter(ne):
    epw = ne // NW
    nit = epw // CH

    @functools.partial(
        pl.kernel,
        out_type=jax.ShapeDtypeStruct((NC, NPAD, H), jnp.float32),
        mesh=_mesh,
        scratch_types=[
            pltpu.VMEM((2, CH), jnp.int32),
            pltpu.VMEM((2, CH, H), jnp.float32),
            pltpu.VMEM_SHARED((NPAD, H), jnp.float32),
            pltpu.SemaphoreType.DMA((2,)),
            pltpu.SemaphoreType.DMA((2,)),
            pltpu.SemaphoreType.DMA((2,)),
        ],
    )
    def scatter(ea_hbm, col_hbm, zeros_hbm, out_hbm, idx_v, buf_v, acc_sh,
                sem, asem, isem):
        cid = lax.axis_index("c")
        sid = lax.axis_index("s")
        wid = sid * NC + cid
        base0 = pl.multiple_of(wid * epw, 8)
        rbase = pl.multiple_of(sid * RPT, 1)

        pltpu.sync_copy(zeros_hbm.at[pl.ds(rbase, RPT)],
                        acc_sh.at[pl.ds(rbase, RPT)])
        plsc.subcore_barrier()

        def start(k, s):
            base = pl.multiple_of(base0 + k * CH, 8)
            pltpu.async_copy(col_hbm.at[pl.ds(base, CH)], idx_v.at[s],
                             isem.at[s])
            pltpu.async_copy(ea_hbm.at[pl.ds(base, CH)], buf_v.at[s],
                             sem.at[s])

        start(0, 0)

        def wait_add(s):
            pltpu.make_async_copy(buf_v.at[s], acc_sh.at[pl.ds(0, CH)],
                                  asem.at[s]).wait()

        def body(it, carry):
            s = lax.rem(it, 2)
            n = 1 - s

            @pl.when(it + 1 < nit)
            def _():
                @pl.when(it >= 1)
                def _():
                    wait_add(n)
                start(it + 1, n)

            pltpu.make_async_copy(ea_hbm.at[pl.ds(0, CH)], buf_v.at[s],
                                  sem.at[s]).wait()
            pltpu.make_async_copy(col_hbm.at[pl.ds(0, CH)], idx_v.at[s],
                                  isem.at[s]).wait()
            pltpu.async_copy(buf_v.at[s], acc_sh.at[idx_v.at[s]], asem.at[s],
                             add=True)
            return carry

        lax.fori_loop(0, nit, body, 0)
        wait_add(lax.rem(nit - 1, 2))
        if nit >= 2:
            wait_add(lax.rem(nit, 2))
        plsc.subcore_barrier()
        pltpu.sync_copy(acc_sh.at[pl.ds(rbase, RPT)],
                        out_hbm.at[cid, pl.ds(rbase, RPT)])

    return scatter


def _make_degree():
    epw = E // NW
    nit = epw // CH

    @functools.partial(
        pl.kernel,
        out_type=jax.ShapeDtypeStruct((NC, NPAD, H), jnp.float32),
        mesh=_mesh,
        scratch_types=[
            pltpu.VMEM((2, CH), jnp.int32),
            pltpu.VMEM((CH, H), jnp.float32),
            pltpu.VMEM_SHARED((NPAD, H), jnp.float32),
            pltpu.SemaphoreType.DMA((2,)),
            pltpu.SemaphoreType.DMA((2,)),
        ],
    )
    def degree(col_hbm, ones_hbm, zeros_hbm, out_hbm, idx_v, buf_v, acc_sh,
               asem, isem):
        cid = lax.axis_index("c")
        sid = lax.axis_index("s")
        wid = sid * NC + cid
        base0 = pl.multiple_of(wid * epw, 8)
        rbase = pl.multiple_of(sid * RPT, 1)

        pltpu.sync_copy(zeros_hbm.at[pl.ds(rbase, RPT)],
                        acc_sh.at[pl.ds(rbase, RPT)])
        pltpu.sync_copy(ones_hbm, buf_v)
        plsc.subcore_barrier()

        def start_idx(k, s):
            base = pl.multiple_of(base0 + k * CH, 8)
            pltpu.async_copy(col_hbm.at[pl.ds(base, CH)], idx_v.at[s],
                             isem.at[s])

        def wait_add(s):
            pltpu.make_async_copy(buf_v, acc_sh.at[pl.ds(0, CH)],
                                  asem.at[s]).wait()

        start_idx(0, 0)

        def body(it, carry):
            s = lax.rem(it, 2)
            n = 1 - s

            @pl.when(it + 1 < nit)
            def _():
                @pl.when(it >= 1)
                def _():
                    wait_add(n)
                start_idx(it + 1, n)

            pltpu.make_async_copy(col_hbm.at[pl.ds(0, CH)], idx_v.at[s],
                                  isem.at[s]).wait()
            pltpu.async_copy(buf_v, acc_sh.at[idx_v.at[s]], asem.at[s],
                             add=True)
            return carry

        lax.fori_loop(0, nit, body, 0)
        wait_add(lax.rem(nit - 1, 2))
        if nit >= 2:
            wait_add(lax.rem(nit, 2))
        plsc.subcore_barrier()
        pltpu.sync_copy(acc_sh.at[pl.ds(rbase, RPT)],
                        out_hbm.at[cid, pl.ds(rbase, RPT)])

    return degree


_gather_a = _make_gather(EA)
_gather_b = _make_gather(EB)
_scatter_a = _make_scatter(EA)
_scatter_b = _make_scatter(EB)
_degree = _make_degree()


BE = 2560
BN = 2000


def _enc_body(x_ref, w_ref, b_ref, g_ref, bt_ref, o_ref):
    t = jnp.dot(x_ref[...], w_ref[...], preferred_element_type=jnp.float32)
    t = t + b_ref[...]
    o_ref[...] = _gelu(_ln(t, g_ref[...], bt_ref[...]))


def _eenc_body(a_ref, w_ref, b_ref, o_ref):
    o_ref[...] = jnp.dot(a_ref[...], w_ref[...],
                         preferred_element_type=jnp.float32) + b_ref[...]


def _edge_body(hr_ref, hc_ref, ea_ref, wr_ref, wc_ref, we_ref, b1_ref,
               g1_ref, bt1_ref, w2_ref, b2_ref, g2_ref, bt2_ref, o_ref):
    ea = ea_ref[...]
    t = (jnp.dot(hr_ref[...].astype(jnp.bfloat16), wr_ref[...],
                 preferred_element_type=jnp.float32)
         + jnp.dot(hc_ref[...].astype(jnp.bfloat16), wc_ref[...],
                   preferred_element_type=jnp.float32)
         + jnp.dot(ea.astype(jnp.bfloat16), we_ref[...],
                   preferred_element_type=jnp.float32)
         + b1_ref[...])
    t = _gelu(_ln(t, g1_ref[...], bt1_ref[...]))
    m = jnp.dot(t.astype(jnp.bfloat16), w2_ref[...],
                preferred_element_type=jnp.float32) + b2_ref[...]
    m = _ln(m, g2_ref[...], bt2_ref[...])
    o_ref[...] = ea + m


def _edge0_body(hr_ref, hc_ref, a_ref, eew_ref, eeb_ref, wr_ref, wc_ref,
                we_ref, b1_ref, g1_ref, bt1_ref, w2_ref, b2_ref, g2_ref,
                bt2_ref, o_ref):
    ea = jnp.dot(a_ref[...], eew_ref[...],
                 preferred_element_type=jnp.float32) + eeb_ref[...]
    t = (jnp.dot(hr_ref[...].astype(jnp.bfloat16), wr_ref[...],
                 preferred_element_type=jnp.float32)
         + jnp.dot(hc_ref[...].astype(jnp.bfloat16), wc_ref[...],
                   preferred_element_type=jnp.float32)
         + jnp.dot(ea.astype(jnp.bfloat16), we_ref[...],
                   preferred_element_type=jnp.float32)
         + b1_ref[...])
    t = _gelu(_ln(t, g1_ref[...], bt1_ref[...]))
    m = jnp.dot(t.astype(jnp.bfloat16), w2_ref[...],
                preferred_element_type=jnp.float32) + b2_ref[...]
    m = _ln(m, g2_ref[...], bt2_ref[...])
    o_ref[...] = ea + m


def _node_body(h_ref, pa_ref, pb_ref, inv_ref, w1a_ref, w1b_ref, b1_ref,
               g1_ref, bt1_ref, w2_ref, b2_ref, g2_ref, bt2_ref, o_ref):
    h = h_ref[...]
    agg = (pa_ref[0] + pa_ref[1] + pb_ref[0] + pb_ref[1]) * inv_ref[...]
    t = (jnp.dot(h, w1a_ref[...], preferred_element_type=jnp.float32)
         + jnp.dot(agg, w1b_ref[...], preferred_element_type=jnp.float32)
         + b1_ref[...])
    t = _gelu(_ln(t, g1_ref[...], bt1_ref[...]))
    u = jnp.dot(t, w2_ref[...], preferred_element_type=jnp.float32) + b2_ref[...]
    u = _ln(u, g2_ref[...], bt2_ref[...])
    o_ref[...] = h + u


def _inv_body(c_ref, o_ref):
    o_ref[...] = 1.0 / jnp.maximum(c_ref[0] + c_ref[1], 1.0)


def _dec_body(h_ref, w1_ref, b1_ref, w2_ref, b2_ref, o_ref):
    t = _gelu(jnp.dot(h_ref[...], w1_ref[...],
                      preferred_element_type=jnp.float32) + b1_ref[...])
    o_ref[...] = jnp.dot(t, w2_ref[...],
                         preferred_element_type=jnp.float32) + b2_ref[...]


def _rowspec(b, w):
    return pl.BlockSpec((b, w), lambda i: (i, 0))


def _wspec(r, c):
    return pl.BlockSpec((r, c), lambda i: (0, 0))


def _tc_call(body, grid, in_specs, out_spec, out_shape):
    return pl.pallas_call(
        body,
        grid=(grid,),
        in_specs=in_specs,
        out_specs=out_spec,
        out_shape=out_shape,
    )


def _edge0_mlp(hr, hc, a, eew, eeb, wr, wc, we, b1, g1, bt1, w2, b2, g2, bt2):
    ne = a.shape[0]
    return _tc_call(
        _edge0_body, ne // BE,
        [_rowspec(BE, H), _rowspec(BE, H), _rowspec(BE, 16),
         _wspec(16, H), _wspec(1, H),
         _wspec(H, 2 * H), _wspec(H, 2 * H), _wspec(H, 2 * H),
         _wspec(1, 2 * H), _wspec(1, 2 * H), _wspec(1, 2 * H),
         _wspec(2 * H, H), _wspec(1, H), _wspec(1, H), _wspec(1, H)],
        _rowspec(BE, H), jax.ShapeDtypeStruct((ne, H), jnp.float32),
    )(hr, hc, a, eew, eeb, wr, wc, we, b1, g1, bt1, w2, b2, g2, bt2)


def _edge_mlp(hr, hc, ea, wr, wc, we, b1, g1, bt1, w2, b2, g2, bt2):
    ne = ea.shape[0]
    return _tc_call(
        _edge_body, ne // BE,
        [_rowspec(BE, H), _rowspec(BE, H), _rowspec(BE, H),
         _wspec(H, 2 * H), _wspec(H, 2 * H), _wspec(H, 2 * H),
         _wspec(1, 2 * H), _wspec(1, 2 * H), _wspec(1, 2 * H),
         _wspec(2 * H, H), _wspec(1, H), _wspec(1, H), _wspec(1, H)],
        _rowspec(BE, H), jax.ShapeDtypeStruct((ne, H), jnp.float32),
    )(hr, hc, ea, wr, wc, we, b1, g1, bt1, w2, b2, g2, bt2)


def kernel(x, edge_index, edge_attr, enc_W, enc_b, enc_g, enc_bt, ee_W, ee_b,
           eW1, eb1, eg1, ebt1, eW2, eb2, eg2, ebt2, nW1, nb1, ng1, nbt1,
           nW2, nb2, ng2, nbt2, dW1, db1, dW2, db2):
    f32 = jnp.float32
    bf16 = jnp.bfloat16
    row = edge_index[0]
    col = edge_index[1]
    row_a, row_b = row[:EA], row[EA:]
    col_a, col_b = col[:EA], col[EA:]
    zeros_n = jnp.zeros((NPAD, H), f32)
    ones_ch = jnp.ones((CH, H), f32)

    h = _tc_call(
        _enc_body, N // BN,
        [_rowspec(BN, H), _wspec(H, H), _wspec(1, H), _wspec(1, H), _wspec(1, H)],
        _rowspec(BN, H), jax.ShapeDtypeStruct((N, H), f32),
    )(x, enc_W, enc_b.reshape(1, H), enc_g.reshape(1, H), enc_bt.reshape(1, H))

    eeb1 = ee_b.reshape(1, H)
    ea_a = None
    ea_b = None

    deg = _degree(col, ones_ch, zeros_n)
    inv_cnt = _tc_call(
        _inv_body, N // BN,
        [pl.BlockSpec((NC, BN, H), lambda i: (0, i, 0))],
        _rowspec(BN, H), jax.ShapeDtypeStruct((N, H), f32),
    )(deg)

    for i in range(eW1.shape[0]):
        w1 = eW1[i]
        wr = w1[:H].astype(bf16)
        wc = w1[H:2 * H].astype(bf16)
        we = w1[2 * H:].astype(bf16)
        eb1i = eb1[i].reshape(1, 2 * H)
        eg1i = eg1[i].reshape(1, 2 * H)
        ebt1i = ebt1[i].reshape(1, 2 * H)
        ew2i = eW2[i].astype(bf16)
        eb2i = eb2[i].reshape(1, H)
        eg2i = eg2[i].reshape(1, H)
        ebt2i = ebt2[i].reshape(1, H)

        hr_a, hc_a = _gather_a(h, row_a, col_a)
        hr_b, hc_b = _gather_b(h, row_b, col_b)
        if i == 0:
            ea_a = _edge0_mlp(hr_a, hc_a, edge_attr[:EA], ee_W, eeb1,
                              wr, wc, we, eb1i, eg1i, ebt1i,
                              ew2i, eb2i, eg2i, ebt2i)
        else:
            ea_a = _edge_mlp(hr_a, hc_a, ea_a, wr, wc, we, eb1i, eg1i, ebt1i,
                             ew2i, eb2i, eg2i, ebt2i)
        pa = _scatter_a(ea_a, col_a, zeros_n)
        if i == 0:
            ea_b = _edge0_mlp(hr_b, hc_b, edge_attr[EA:], ee_W, eeb1,
                              wr, wc, we, eb1i, eg1i, ebt1i,
                              ew2i, eb2i, eg2i, ebt2i)
        else:
            ea_b = _edge_mlp(hr_b, hc_b, ea_b, wr, wc, we, eb1i, eg1i, ebt1i,
                             ew2i, eb2i, eg2i, ebt2i)
        pb = _scatter_b(ea_b, col_b, zeros_n)

        nw1 = nW1[i]
        h = _tc_call(
            _node_body, N // BN,
            [_rowspec(BN, H),
             pl.BlockSpec((NC, BN, H), lambda i: (0, i, 0)),
             pl.BlockSpec((NC, BN, H), lambda i: (0, i, 0)),
             _rowspec(BN, H),
             _wspec(H, 2 * H), _wspec(H, 2 * H),
             _wspec(1, 2 * H), _wspec(1, 2 * H), _wspec(1, 2 * H),
             _wspec(2 * H, H), _wspec(1, H), _wspec(1, H), _wspec(1, H)],
            _rowspec(BN, H), jax.ShapeDtypeStruct((N, H), f32),
        )(h, pa, pb, inv_cnt, nw1[:H], nw1[H:],
          nb1[i].reshape(1, 2 * H), ng1[i].reshape(1, 2 * H),
          nbt1[i].reshape(1, 2 * H), nW2[i], nb2[i].reshape(1, H),
          ng2[i].reshape(1, H), nbt2[i].reshape(1, H))

    dW2p = jnp.zeros((H, H), f32).at[:, :dW2.shape[1]].set(dW2)
    db2p = jnp.zeros((1, H), f32).at[0, :db2.shape[0]].set(db2)
    out = _tc_call(
        _dec_body, N // BN,
        [_rowspec(BN, H), _wspec(H, H), _wspec(1, H), _wspec(H, H), _wspec(1, H)],
        _rowspec(BN, H), jax.ShapeDtypeStruct((N, H), f32),
    )(h, dW1, db1.reshape(1, H), dW2p, db2p)
    return out[:, :dW2.shape[1]]


# speedup vs baseline: 5.7322x; 1.0451x over previous
import functools

import jax
import jax.numpy as jnp
from jax import lax
from jax.experimental import pallas as pl
from jax.experimental.pallas import tpu as pltpu
from jax.experimental.pallas import tpu_sc as plsc

N = 10000
E = 320000
H = 128

NC = 2
NS = 16
NW = NC * NS
CH = 80
NPAD = 10240
RPT = NPAD // NS

PH_SIZES = (81920, 81920, 81920, 74240)
PH_OFF = (0, 81920, 163840, 245760)

_mesh = plsc.VectorSubcoreMesh(
    core_axis_name="c", subcore_axis_name="s", num_cores=NC, num_subcores=NS)


def _ln(t, g, b):
    mu = jnp.mean(t, axis=-1, keepdims=True)
    t = t - mu
    var = jnp.mean(t * t, axis=-1, keepdims=True)
    return t * lax.rsqrt(var + 1e-5) * g + b


def _gelu(t):
    return t * 0.5 * (1.0 + lax.erf(t * 0.7071067811865476))


def _make_gather(ne):
    epw = ne // NW
    nit = epw // CH

    @functools.partial(
        pl.kernel,
        out_type=(jax.ShapeDtypeStruct((ne, H), jnp.float32),
                  jax.ShapeDtypeStruct((ne, H), jnp.float32)),
        mesh=_mesh,
        scratch_types=[
            pltpu.VMEM((2, CH), jnp.int32),
            pltpu.VMEM((2, CH), jnp.int32),
            pltpu.VMEM((2, CH, H), jnp.float32),
            pltpu.VMEM((2, CH, H), jnp.float32),
            pltpu.SemaphoreType.DMA((2,)),
            pltpu.SemaphoreType.DMA((2,)),
            pltpu.SemaphoreType.DMA((2,)),
            pltpu.SemaphoreType.DMA((2,)),
            pltpu.SemaphoreType.DMA((2,)),
            pltpu.SemaphoreType.DMA((2,)),
        ],
    )
    def gather(h_hbm, row_hbm, col_hbm, hr_hbm, hc_hbm,
               idx_r, idx_c, buf_r, buf_c, sem_r, sem_c, wsem_r, wsem_c,
               isem_r, isem_c):
        wid = lax.axis_index("s") * NC + lax.axis_index("c")
        base0 = pl.multiple_of(wid * epw, 8)

        def start_idx(k, s):
            base = pl.multiple_of(base0 + k * CH, 8)
            pltpu.async_copy(row_hbm.at[pl.ds(base, CH)], idx_r.at[s],
                             isem_r.at[s])
            pltpu.async_copy(col_hbm.at[pl.ds(base, CH)], idx_c.at[s],
                             isem_c.at[s])

        def wait_idx(s):
            pltpu.make_async_copy(row_hbm.at[pl.ds(0, CH)], idx_r.at[s],
                                  isem_r.at[s]).wait()
            pltpu.make_async_copy(col_hbm.at[pl.ds(0, CH)], idx_c.at[s],
                                  isem_c.at[s]).wait()

        def start_gather(s):
            pltpu.async_copy(h_hbm.at[idx_r.at[s]], buf_r.at[s], sem_r.at[s])
            pltpu.async_copy(h_hbm.at[idx_c.at[s]], buf_c.at[s], sem_c.at[s])

        def wait_wb(s):
            pltpu.make_async_copy(buf_r.at[s], hr_hbm.at[pl.ds(0, CH)],
                                  wsem_r.at[s]).wait()
            pltpu.make_async_copy(buf_c.at[s], hc_hbm.at[pl.ds(0, CH)],
                                  wsem_c.at[s]).wait()

        start_idx(0, 0)
        if nit > 1:
            start_idx(1, 1)
        wait_idx(0)
        start_gather(0)

        def body(it, carry):
            s = lax.rem(it, 2)
            n = 1 - s

            @pl.when(it + 1 < nit)
            def _():
                @pl.when(it >= 1)
                def _():
                    wait_wb(n)
                wait_idx(n)
                start_gather(n)

            base = pl.multiple_of(base0 + it * CH, 8)
            pltpu.make_async_copy(hr_hbm.at[pl.ds(0, CH)], buf_r.at[s],
                                  sem_r.at[s]).wait()
            pltpu.make_async_copy(hr_hbm.at[pl.ds(0, CH)], buf_c.at[s],
                                  sem_c.at[s]).wait()

            @pl.when(it + 2 < nit)
            def _():
                start_idx(it + 2, s)

            pltpu.async_copy(buf_r.at[s], hr_hbm.at[pl.ds(base, CH)],
                             wsem_r.at[s])
            pltpu.async_copy(buf_c.at[s], hc_hbm.at[pl.ds(base, CH)],
                             wsem_c.at[s])
            return carry

        lax.fori_loop(0, nit, body, 0)
        wait_wb(lax.rem(nit - 1, 2))
        if nit >= 2:
            wait_wb(lax.rem(nit, 2))

    return gather


def _make_scatter(ne):
    epw = ne // NW
    nit = epw // CH

    @functools.partial(
        pl.kernel,
        out_type=jax.ShapeDtypeStruct((NC, NPAD, H), jnp.float32),
        mesh=_mesh,
        scratch_types=[
            pltpu.VMEM((2, CH), jnp.int32),
            pltpu.VMEM((2, CH, H), jnp.float32),
            pltpu.VMEM_SHARED((NPAD, H), jnp.float32),
            pltpu.SemaphoreType.DMA((2,)),
            pltpu.SemaphoreType.DMA((2,)),
            pltpu.SemaphoreType.DMA((2,)),
        ],
    )
    def scatter(ea_hbm, col_hbm, zeros_hbm, out_hbm, idx_v, buf_v, acc_sh,
                sem, asem, isem):
        cid = lax.axis_index("c")
        sid = lax.axis_index("s")
        wid = sid * NC + cid
        base0 = pl.multiple_of(wid * epw, 8)
        rbase = pl.multiple_of(sid * RPT, 1)

        pltpu.sync_copy(zeros_hbm.at[pl.ds(rbase, RPT)],
                        acc_sh.at[pl.ds(rbase, RPT)])
        plsc.subcore_barrier()

        def start(k, s):
            base = pl.multiple_of(base0 + k * CH, 8)
            pltpu.async_copy(col_hbm.at[pl.ds(base, CH)], idx_v.at[s],
                             isem.at[s])
            pltpu.async_copy(ea_hbm.at[pl.ds(base, CH)], buf_v.at[s],
                             sem.at[s])

        start(0, 0)

        def wait_add(s):
            pltpu.make_async_copy(buf_v.at[s], acc_sh.at[pl.ds(0, CH)],
                                  asem.at[s]).wait()

        def body(it, carry):
            s = lax.rem(it, 2)
            n = 1 - s

            @pl.when(it + 1 < nit)
            def _():
                @pl.when(it >= 1)
                def _():
                    wait_add(n)
                start(it + 1, n)

            pltpu.make_async_copy(ea_hbm.at[pl.ds(0, CH)], buf_v.at[s],
                                  sem.at[s]).wait()
            pltpu.make_async_copy(col_hbm.at[pl.ds(0, CH)], idx_v.at[s],
                                  isem.at[s]).wait()
            pltpu.async_copy(buf_v.at[s], acc_sh.at[idx_v.at[s]], asem.at[s],
                             add=True)
            return carry

        lax.fori_loop(0, nit, body, 0)
        wait_add(lax.rem(nit - 1, 2))
        if nit >= 2:
            wait_add(lax.rem(nit, 2))
        plsc.subcore_barrier()
        pltpu.sync_copy(acc_sh.at[pl.ds(rbase, RPT)],
                        out_hbm.at[cid, pl.ds(rbase, RPT)])

    return scatter


def _make_degree():
    epw = E // NW
    nit = epw // CH

    @functools.partial(
        pl.kernel,
        out_type=jax.ShapeDtypeStruct((NC, NPAD, H), jnp.float32),
        mesh=_mesh,
        scratch_types=[
            pltpu.VMEM((2, CH), jnp.int32),
            pltpu.VMEM((CH, H), jnp.float32),
            pltpu.VMEM_SHARED((NPAD, H), jnp.float32),
            pltpu.SemaphoreType.DMA((2,)),
            pltpu.SemaphoreType.DMA((2,)),
        ],
    )
    def degree(col_hbm, ones_hbm, zeros_hbm, out_hbm, idx_v, buf_v, acc_sh,
               asem, isem):
        cid = lax.axis_index("c")
        sid = lax.axis_index("s")
        wid = sid * NC + cid
        base0 = pl.multiple_of(wid * epw, 8)
        rbase = pl.multiple_of(sid * RPT, 1)

        pltpu.sync_copy(zeros_hbm.at[pl.ds(rbase, RPT)],
                        acc_sh.at[pl.ds(rbase, RPT)])
        pltpu.sync_copy(ones_hbm, buf_v)
        plsc.subcore_barrier()

        def start_idx(k, s):
            base = pl.multiple_of(base0 + k * CH, 8)
            pltpu.async_copy(col_hbm.at[pl.ds(base, CH)], idx_v.at[s],
                             isem.at[s])

        def wait_add(s):
            pltpu.make_async_copy(buf_v, acc_sh.at[pl.ds(0, CH)],
                                  asem.at[s]).wait()

        start_idx(0, 0)

        def body(it, carry):
            s = lax.rem(it, 2)
            n = 1 - s

            @pl.when(it + 1 < nit)
            def _():
                @pl.when(it >= 1)
                def _():
                    wait_add(n)
                start_idx(it + 1, n)

            pltpu.make_async_copy(col_hbm.at[pl.ds(0, CH)], idx_v.at[s],
                                  isem.at[s]).wait()
            pltpu.async_copy(buf_v, acc_sh.at[idx_v.at[s]], asem.at[s],
                             add=True)
            return carry

        lax.fori_loop(0, nit, body, 0)
        wait_add(lax.rem(nit - 1, 2))
        if nit >= 2:
            wait_add(lax.rem(nit, 2))
        plsc.subcore_barrier()
        pltpu.sync_copy(acc_sh.at[pl.ds(rbase, RPT)],
                        out_hbm.at[cid, pl.ds(rbase, RPT)])

    return degree


_gathers = {ne: _make_gather(ne) for ne in set(PH_SIZES)}
_scatters = {ne: _make_scatter(ne) for ne in set(PH_SIZES)}
_degree = _make_degree()


BE = 2560
BN = 2000


def _enc_body(x_ref, w_ref, b_ref, g_ref, bt_ref, o_ref):
    t = jnp.dot(x_ref[...], w_ref[...], preferred_element_type=jnp.float32)
    t = t + b_ref[...]
    o_ref[...] = _gelu(_ln(t, g_ref[...], bt_ref[...]))


def _eenc_body(a_ref, w_ref, b_ref, o_ref):
    o_ref[...] = jnp.dot(a_ref[...], w_ref[...],
                         preferred_element_type=jnp.float32) + b_ref[...]


def _edge_body(hr_ref, hc_ref, ea_ref, wr_ref, wc_ref, we_ref, b1_ref,
               g1_ref, bt1_ref, w2_ref, b2_ref, g2_ref, bt2_ref, o_ref):
    ea = ea_ref[...]
    t = (jnp.dot(hr_ref[...].astype(jnp.bfloat16), wr_ref[...],
                 preferred_element_type=jnp.float32)
         + jnp.dot(hc_ref[...].astype(jnp.bfloat16), wc_ref[...],
                   preferred_element_type=jnp.float32)
         + jnp.dot(ea.astype(jnp.bfloat16), we_ref[...],
                   preferred_element_type=jnp.float32)
         + b1_ref[...])
    t = _gelu(_ln(t, g1_ref[...], bt1_ref[...]))
    m = jnp.dot(t.astype(jnp.bfloat16), w2_ref[...],
                preferred_element_type=jnp.float32) + b2_ref[...]
    m = _ln(m, g2_ref[...], bt2_ref[...])
    o_ref[...] = ea + m


def _edge0_body(hr_ref, hc_ref, a_ref, eew_ref, eeb_ref, wr_ref, wc_ref,
                we_ref, b1_ref, g1_ref, bt1_ref, w2_ref, b2_ref, g2_ref,
                bt2_ref, o_ref):
    ea = jnp.dot(a_ref[...], eew_ref[...],
                 preferred_element_type=jnp.float32) + eeb_ref[...]
    t = (jnp.dot(hr_ref[...].astype(jnp.bfloat16), wr_ref[...],
                 preferred_element_type=jnp.float32)
         + jnp.dot(hc_ref[...].astype(jnp.bfloat16), wc_ref[...],
                   preferred_element_type=jnp.float32)
         + jnp.dot(ea.astype(jnp.bfloat16), we_ref[...],
                   preferred_element_type=jnp.float32)
         + b1_ref[...])
    t = _gelu(_ln(t, g1_ref[...], bt1_ref[...]))
    m = jnp.dot(t.astype(jnp.bfloat16), w2_ref[...],
                preferred_element_type=jnp.float32) + b2_ref[...]
    m = _ln(m, g2_ref[...], bt2_ref[...])
    o_ref[...] = ea + m


def _node_body(h_ref, p0_ref, p1_ref, p2_ref, p3_ref, inv_ref, w1a_ref,
               w1b_ref, b1_ref, g1_ref, bt1_ref, w2_ref, b2_ref, g2_ref,
               bt2_ref, o_ref):
    h = h_ref[...]
    agg = (p0_ref[0] + p0_ref[1] + p1_ref[0] + p1_ref[1]
           + p2_ref[0] + p2_ref[1] + p3_ref[0] + p3_ref[1]) * inv_ref[...]
    t = (jnp.dot(h, w1a_ref[...], preferred_element_type=jnp.float32)
         + jnp.dot(agg, w1b_ref[...], preferred_element_type=jnp.float32)
         + b1_ref[...])
    t = _gelu(_ln(t, g1_ref[...], bt1_ref[...]))
    u = jnp.dot(t, w2_ref[...], preferred_element_type=jnp.float32) + b2_ref[...]
    u = _ln(u, g2_ref[...], bt2_ref[...])
    o_ref[...] = h + u


def _inv_body(c_ref, o_ref):
    o_ref[...] = 1.0 / jnp.maximum(c_ref[0] + c_ref[1], 1.0)


def _dec_body(h_ref, w1_ref, b1_ref, w2_ref, b2_ref, o_ref):
    t = _gelu(jnp.dot(h_ref[...], w1_ref[...],
                      preferred_element_type=jnp.float32) + b1_ref[...])
    o_ref[...] = jnp.dot(t, w2_ref[...],
                         preferred_element_type=jnp.float32) + b2_ref[...]


def _rowspec(b, w):
    return pl.BlockSpec((b, w), lambda i: (i, 0))


def _wspec(r, c):
    return pl.BlockSpec((r, c), lambda i: (0, 0))


def _tc_call(body, grid, in_specs, out_spec, out_shape):
    return pl.pallas_call(
        body,
        grid=(grid,),
        in_specs=in_specs,
        out_specs=out_spec,
        out_shape=out_shape,
    )


def _edge0_mlp(hr, hc, a, eew, eeb, wr, wc, we, b1, g1, bt1, w2, b2, g2, bt2):
    ne = a.shape[0]
    return _tc_call(
        _edge0_body, ne // BE,
        [_rowspec(BE, H), _rowspec(BE, H), _rowspec(BE, 16),
         _wspec(16, H), _wspec(1, H),
         _wspec(H, 2 * H), _wspec(H, 2 * H), _wspec(H, 2 * H),
         _wspec(1, 2 * H), _wspec(1, 2 * H), _wspec(1, 2 * H),
         _wspec(2 * H, H), _wspec(1, H), _wspec(1, H), _wspec(1, H)],
        _rowspec(BE, H), jax.ShapeDtypeStruct((ne, H), jnp.float32),
    )(hr, hc, a, eew, eeb, wr, wc, we, b1, g1, bt1, w2, b2, g2, bt2)


def _edge_mlp(hr, hc, ea, wr, wc, we, b1, g1, bt1, w2, b2, g2, bt2):
    ne = ea.shape[0]
    return _tc_call(
        _edge_body, ne // BE,
        [_rowspec(BE, H), _rowspec(BE, H), _rowspec(BE, H),
         _wspec(H, 2 * H), _wspec(H, 2 * H), _wspec(H, 2 * H),
         _wspec(1, 2 * H), _wspec(1, 2 * H), _wspec(1, 2 * H),
         _wspec(2 * H, H), _wspec(1, H), _wspec(1, H), _wspec(1, H)],
        _rowspec(BE, H), jax.ShapeDtypeStruct((ne, H), jnp.float32),
    )(hr, hc, ea, wr, wc, we, b1, g1, bt1, w2, b2, g2, bt2)


def kernel(x, edge_index, edge_attr, enc_W, enc_b, enc_g, enc_bt, ee_W, ee_b,
           eW1, eb1, eg1, ebt1, eW2, eb2, eg2, ebt2, nW1, nb1, ng1, nbt1,
           nW2, nb2, ng2, nbt2, dW1, db1, dW2, db2):
    f32 = jnp.float32
    bf16 = jnp.bfloat16
    row = edge_index[0]
    col = edge_index[1]
    rows = [row[o:o + n] for o, n in zip(PH_OFF, PH_SIZES)]
    cols = [col[o:o + n] for o, n in zip(PH_OFF, PH_SIZES)]
    attrs = [edge_attr[o:o + n] for o, n in zip(PH_OFF, PH_SIZES)]
    zeros_n = jnp.zeros((NPAD, H), f32)
    ones_ch = jnp.ones((CH, H), f32)

    h = _tc_call(
        _enc_body, N // BN,
        [_rowspec(BN, H), _wspec(H, H), _wspec(1, H), _wspec(1, H), _wspec(1, H)],
        _rowspec(BN, H), jax.ShapeDtypeStruct((N, H), f32),
    )(x, enc_W, enc_b.reshape(1, H), enc_g.reshape(1, H), enc_bt.reshape(1, H))

    eeb1 = ee_b.reshape(1, H)
    eas = [None] * len(PH_SIZES)

    deg = _degree(col, ones_ch, zeros_n)
    inv_cnt = _tc_call(
        _inv_body, N // BN,
        [pl.BlockSpec((NC, BN, H), lambda i: (0, i, 0))],
        _rowspec(BN, H), jax.ShapeDtypeStruct((N, H), f32),
    )(deg)

    for i in range(eW1.shape[0]):
        w1 = eW1[i]
        wr = w1[:H].astype(bf16)
        wc = w1[H:2 * H].astype(bf16)
        we = w1[2 * H:].astype(bf16)
        eb1i = eb1[i].reshape(1, 2 * H)
        eg1i = eg1[i].reshape(1, 2 * H)
        ebt1i = ebt1[i].reshape(1, 2 * H)
        ew2i = eW2[i].astype(bf16)
        eb2i = eb2[i].reshape(1, H)
        eg2i = eg2[i].reshape(1, H)
        ebt2i = ebt2[i].reshape(1, H)

        ghs = [_gathers[n](h, r, c)
               for n, r, c in zip(PH_SIZES, rows, cols)]
        parts = []
        for p, (hr_p, hc_p) in enumerate(ghs):
            if i == 0:
                eas[p] = _edge0_mlp(hr_p, hc_p, attrs[p], ee_W, eeb1,
                                    wr, wc, we, eb1i, eg1i, ebt1i,
                                    ew2i, eb2i, eg2i, ebt2i)
            else:
                eas[p] = _edge_mlp(hr_p, hc_p, eas[p], wr, wc, we, eb1i,
                                   eg1i, ebt1i, ew2i, eb2i, eg2i, ebt2i)
            parts.append(_scatters[PH_SIZES[p]](eas[p], cols[p], zeros_n))

        nw1 = nW1[i]
        h = _tc_call(
            _node_body, N // BN,
            [_rowspec(BN, H),
             pl.BlockSpec((NC, BN, H), lambda i: (0, i, 0)),
             pl.BlockSpec((NC, BN, H), lambda i: (0, i, 0)),
             pl.BlockSpec((NC, BN, H), lambda i: (0, i, 0)),
             pl.BlockSpec((NC, BN, H), lambda i: (0, i, 0)),
             _rowspec(BN, H),
             _wspec(H, 2 * H), _wspec(H, 2 * H),
             _wspec(1, 2 * H), _wspec(1, 2 * H), _wspec(1, 2 * H),
             _wspec(2 * H, H), _wspec(1, H), _wspec(1, H), _wspec(1, H)],
            _rowspec(BN, H), jax.ShapeDtypeStruct((N, H), f32),
        )(h, parts[0], parts[1], parts[2], parts[3], inv_cnt,
          nw1[:H], nw1[H:],
          nb1[i].reshape(1, 2 * H), ng1[i].reshape(1, 2 * H),
          nbt1[i].reshape(1, 2 * H), nW2[i], nb2[i].reshape(1, H),
          ng2[i].reshape(1, H), nbt2[i].reshape(1, H))

    dW2p = jnp.zeros((H, H), f32).at[:, :dW2.shape[1]].set(dW2)
    db2p = jnp.zeros((1, H), f32).at[0, :db2.shape[0]].set(db2)
    out = _tc_call(
        _dec_body, N // BN,
        [_rowspec(BN, H), _wspec(H, H), _wspec(1, H), _wspec(H, H), _wspec(1, H)],
        _rowspec(BN, H), jax.ShapeDtypeStruct((N, H), f32),
    )(h, dW1, db1.reshape(1, H), dW2p, db2p)
    return out[:, :dW2.shape[1]]
